# Initial kernel scaffold; baseline (speedup 1.0000x reference)
import jax
import jax.numpy as jnp
from jax.experimental import pallas as pl


def kernel(x, edge_index, edge_label_index, Wl1, bl1, Wr1, Wl2, bl2, Wr2):
    raise NotImplementedError("write your pallas kernel here")


# speedup vs baseline: 2.1792x; 2.1792x over previous
import functools

import jax
import jax.numpy as jnp
from jax import lax
from jax.experimental import pallas as pl
from jax.experimental.pallas import tpu as pltpu
from jax.experimental.pallas import tpu_sc as plsc

N = 10000
E = 320000
D = 128

NC = 2
NS = 16
NW = NC * NS
CH = 80
CPT = E // (NW * CH)
NPAD = 10240
RPT = NPAD // NS

BM = 1000


def _mm_aug(x, w):
    def body(x_ref, w_ref, o_ref):
        y = lax.dot_general(x_ref[...], w_ref[...], (((1,), (1,)), ((), ())),
                            preferred_element_type=jnp.float32)
        ones = jnp.ones((BM, 1), jnp.float32)
        zer = jnp.zeros((BM, 15), jnp.float32)
        o_ref[...] = jnp.concatenate([y, ones, zer], axis=1)

    return pl.pallas_call(
        body,
        grid=(N // BM,),
        in_specs=[pl.BlockSpec((BM, D), lambda i: (i, 0)),
                  pl.BlockSpec((D, D), lambda i: (0, 0))],
        out_specs=pl.BlockSpec((BM, D + 16), lambda i: (i, 0)),
        out_shape=jax.ShapeDtypeStruct((N, D + 16), jnp.float32),
    )(x, w)


def _combine1(a0, a1, x, wr, bl, wl2):
    def body(a0_ref, a1_ref, x_ref, wr_ref, bl_ref, wl2_ref,
             h_ref, y2_ref, inv_ref):
        s = a0_ref[...] + a1_ref[...]
        inv = 1.0 / jnp.maximum(s[:, D:D + 1], 1.0)
        lin = lax.dot_general(x_ref[...], wr_ref[...], (((1,), (1,)), ((), ())),
                              preferred_element_type=jnp.float32)
        h = jnp.maximum(s[:, :D] * inv + bl_ref[...] + lin, 0.0)
        h_ref[...] = h
        y2_ref[...] = lax.dot_general(h, wl2_ref[...], (((1,), (1,)), ((), ())),
                                      preferred_element_type=jnp.float32)
        inv_ref[...] = inv

    return pl.pallas_call(
        body,
        grid=(N // BM,),
        in_specs=[pl.BlockSpec((BM, D + 16), lambda i: (i, 0)),
                  pl.BlockSpec((BM, D + 16), lambda i: (i, 0)),
                  pl.BlockSpec((BM, D), lambda i: (i, 0)),
                  pl.BlockSpec((D, D), lambda i: (0, 0)),
                  pl.BlockSpec((1, D), lambda i: (0, 0)),
                  pl.BlockSpec((D, D), lambda i: (0, 0))],
        out_specs=[pl.BlockSpec((BM, D), lambda i: (i, 0)),
                   pl.BlockSpec((BM, D), lambda i: (i, 0)),
                   pl.BlockSpec((BM, 1), lambda i: (i, 0))],
        out_shape=[jax.ShapeDtypeStruct((N, D), jnp.float32),
                   jax.ShapeDtypeStruct((N, D), jnp.float32),
                   jax.ShapeDtypeStruct((N, 1), jnp.float32)],
    )(a0, a1, x, wr, bl, wl2)


def _combine2(a0, a1, h, wr, bl, inv):
    def body(a0_ref, a1_ref, h_ref, wr_ref, bl_ref, inv_ref, z_ref):
        s = a0_ref[...] + a1_ref[...]
        lin = lax.dot_general(h_ref[...], wr_ref[...], (((1,), (1,)), ((), ())),
                              preferred_element_type=jnp.float32)
        z_ref[...] = s * inv_ref[...] + bl_ref[...] + lin

    return pl.pallas_call(
        body,
        grid=(N // BM,),
        in_specs=[pl.BlockSpec((BM, D), lambda i: (i, 0)),
                  pl.BlockSpec((BM, D), lambda i: (i, 0)),
                  pl.BlockSpec((BM, D), lambda i: (i, 0)),
                  pl.BlockSpec((D, D), lambda i: (0, 0)),
                  pl.BlockSpec((1, D), lambda i: (0, 0)),
                  pl.BlockSpec((BM, 1), lambda i: (i, 0))],
        out_specs=pl.BlockSpec((BM, D), lambda i: (i, 0)),
        out_shape=jax.ShapeDtypeStruct((N, D), jnp.float32),
    )(a0, a1, h, wr, bl, inv)


def _make_agg(width):
    mesh = plsc.VectorSubcoreMesh(core_axis_name="c", subcore_axis_name="s")

    def body(y, src2d, dst2d, zeros, out, sidx, didx, rows, accum, sem):
        cid = lax.axis_index("c")
        sid = lax.axis_index("s")
        wid = sid * NC + cid
        pltpu.sync_copy(zeros.at[pl.ds(sid * RPT, RPT)],
                        accum.at[pl.ds(sid * RPT, RPT)])
        plsc.subcore_barrier()

        def chunk(j, carry):
            row = wid * CPT + j
            pltpu.sync_copy(src2d.at[row], sidx)
            pltpu.sync_copy(dst2d.at[row], didx)
            pltpu.async_copy(y.at[sidx], rows, sem).wait()
            pltpu.sync_copy(rows, accum.at[didx], add=True)
            return carry

        lax.fori_loop(0, CPT, chunk, 0)
        plsc.subcore_barrier()
        pltpu.sync_copy(accum.at[pl.ds(sid * RPT, RPT)],
                        out.at[cid, pl.ds(sid * RPT, RPT)])

    return pl.kernel(
        body,
        out_type=jax.ShapeDtypeStruct((NC, NPAD, width), jnp.float32),
        mesh=mesh,
        compiler_params=pltpu.CompilerParams(use_tc_tiling_on_sc=False),
        scratch_types=[
            pltpu.VMEM((CH,), jnp.int32),
            pltpu.VMEM((CH,), jnp.int32),
            pltpu.VMEM((CH, width), jnp.float32),
            pltpu.VMEM_SHARED((NPAD, width), jnp.float32),
            pltpu.SemaphoreType.DMA,
        ],
    )


def _make_decode():
    mesh = plsc.VectorSubcoreMesh(core_axis_name="c", subcore_axis_name="s")

    def body(z, s2d, d2d, out, ia, ib, av, bv, ov, sema, semb):
        cid = lax.axis_index("c")
        sid = lax.axis_index("s")
        wid = sid * NC + cid

        def chunk(j, carry):
            row = wid * CPT + j
            pltpu.sync_copy(s2d.at[row], ia)
            pltpu.sync_copy(d2d.at[row], ib)
            ca = pltpu.async_copy(z.at[ia], av, sema)
            cb = pltpu.async_copy(z.at[ib], bv, semb)
            ca.wait()
            cb.wait()
            for g in range(CH // 16):
                eids = jnp.full((16,), g * 16, jnp.int32) + lax.iota(jnp.int32, 16)

                def fbody(f, acc):
                    fv = jnp.full((16,), f, jnp.int32)
                    ga = plsc.load_gather(av, [eids, fv])
                    gb = plsc.load_gather(bv, [eids, fv])
                    return acc + ga * gb

                acc = lax.fori_loop(0, D, fbody, jnp.zeros((16,), jnp.float32))
                ov[pl.ds(g * 16, 16)] = acc
            pltpu.sync_copy(ov, out.at[pl.ds(row * CH, CH)])
            return carry

        lax.fori_loop(0, CPT, chunk, 0)

    return pl.kernel(
        body,
        out_type=jax.ShapeDtypeStruct((E,), jnp.float32),
        mesh=mesh,
        compiler_params=pltpu.CompilerParams(use_tc_tiling_on_sc=False,
                                             needs_layout_passes=False),
        scratch_types=[
            pltpu.VMEM((CH,), jnp.int32),
            pltpu.VMEM((CH,), jnp.int32),
            pltpu.VMEM((CH, D), jnp.float32),
            pltpu.VMEM((CH, D), jnp.float32),
            pltpu.VMEM((CH,), jnp.float32),
            pltpu.SemaphoreType.DMA,
            pltpu.SemaphoreType.DMA,
        ],
    )


_agg_aug = _make_agg(D + 16)
_agg_plain = _make_agg(D)
_decode = _make_decode()


def kernel(x, edge_index, edge_label_index, Wl1, bl1, Wr1, Wl2, bl2, Wr2):
    src2d = edge_index[0].astype(jnp.int32).reshape(E // CH, CH)
    dst2d = edge_index[1].astype(jnp.int32).reshape(E // CH, CH)
    es2d = edge_label_index[0].astype(jnp.int32).reshape(E // CH, CH)
    ed2d = edge_label_index[1].astype(jnp.int32).reshape(E // CH, CH)
    zeros_aug = jnp.zeros((NPAD, D + 16), jnp.float32)
    zeros_pln = jnp.zeros((NPAD, D), jnp.float32)

    y1 = _mm_aug(x, Wl1)
    agg1 = _agg_aug(y1, src2d, dst2d, zeros_aug)
    h, y2, inv = _combine1(agg1[0], agg1[1], x, Wr1, bl1.reshape(1, D), Wl2)

    agg2 = _agg_plain(y2, src2d, dst2d, zeros_pln)
    z = _combine2(agg2[0], agg2[1], h, Wr2, bl2.reshape(1, D), inv)

    return _decode(z, es2d, ed2d)
